# baseline (speedup 1.0000x reference)
import functools
import math

import jax
import jax.numpy as jnp
from jax import lax
from jax.experimental import pallas as pl
from jax.experimental.pallas import tpu as pltpu
from jax.experimental.pallas import tpu_sc as plsc

SIZE = 32000
PAD_IDX = 0
N_TOKENS = 4096

_SMOOTH = 0.1
_CONF = 1.0 - _SMOOTH
_EPS = _SMOOTH / (SIZE - 2)
_C_ROW = (SIZE - 2) * _EPS * math.log(_EPS) + _CONF * math.log(_CONF)

L = 16
NC = 2
NS = 16
NW = NC * NS
RPW = N_TOKENS // NW
NCH = RPW // L
_U = 16
_KITERS = SIZE // (L * _U)

_GDN = lax.GatherDimensionNumbers(
    offset_dims=(), collapsed_slice_dims=(0,), start_index_map=(0,))


def _bcast_lane(vec, lane):
    idx = jnp.full((L, 1), lane, jnp.int32)
    return lax.gather(vec, idx, _GDN, slice_sizes=(1,),
                      mode=lax.GatherScatterMode.PROMISE_IN_BOUNDS)


@functools.lru_cache(maxsize=1)
def _build_sc_loss():
    mesh = plsc.VectorSubcoreMesh(
        core_axis_name="c", subcore_axis_name="s",
        num_cores=NC, num_subcores=NS,
    )

    @functools.partial(
        pl.kernel,
        out_type=jax.ShapeDtypeStruct((NW, L), jnp.float32),
        mesh=mesh,
        scratch_types=[
            pltpu.VMEM((RPW,), jnp.int32),
            pltpu.VMEM((RPW,), jnp.int32),
            pltpu.VMEM((RPW,), jnp.int32),
            pltpu.VMEM((RPW,), jnp.float32),
            pltpu.VMEM((RPW,), jnp.float32),
            pltpu.VMEM((RPW,), jnp.float32),
            pltpu.VMEM((SIZE,), jnp.float32),
            pltpu.VMEM((SIZE,), jnp.float32),
            pltpu.VMEM((L,), jnp.float32),
            pltpu.SemaphoreType.DMA,
            pltpu.SemaphoreType.DMA,
            pltpu.SemaphoreType.DMA,
            pltpu.SemaphoreType.DMA,
        ],
    )
    def sc_loss(xf_hbm, t_hbm, out_hbm,
                t_v, it_v, i0_v, gt_v, g0_v, mf_v, buf0, buf1, acc_v,
                sem_t, sem_0, sem_b0, sem_b1):
        cid = lax.axis_index("c")
        sid = lax.axis_index("s")
        wid = cid * NS + sid
        base = wid * RPW

        pltpu.sync_copy(t_hbm.at[pl.ds(base, RPW)], t_v)

        for c in range(NCH):
            t = t_v[pl.ds(c * L, L)]
            rows = (base + c * L) + lax.iota(jnp.int32, L)
            flat0 = rows * SIZE
            it_v[pl.ds(c * L, L)] = flat0 + t
            i0_v[pl.ds(c * L, L)] = flat0
            mf_v[pl.ds(c * L, L)] = jnp.where(t != PAD_IDX, jnp.float32(1.0),
                                              jnp.float32(0.0))

        cp_t = pltpu.async_copy(xf_hbm.at[it_v], gt_v, sem_t)
        cp_0 = pltpu.async_copy(xf_hbm.at[i0_v], g0_v, sem_0)

        bufs = (buf0, buf1)
        sems = (sem_b0, sem_b1)

        def start_row(row, b):
            @pl.when(row < RPW)
            def _():
                pltpu.async_copy(
                    xf_hbm.at[pl.ds((base + row) * SIZE, SIZE)],
                    bufs[b], sems[b])

        def wait_buf(b):
            pltpu.make_async_copy(xf_hbm.at[pl.ds(0, SIZE)],
                                  bufs[b], sems[b]).wait()

        def reduce_row(b):
            def step(k, accs):
                new = list(accs)
                for u in range(_U):
                    v = bufs[b][pl.ds(k * (L * _U) + u * L, L)]
                    new[u % 4] = new[u % 4] + v
                return tuple(new)

            z = jnp.zeros((L,), jnp.float32)
            a0, a1, a2, a3 = lax.fori_loop(0, _KITERS, step, (z, z, z, z))
            return (a0 + a1) + (a2 + a3)

        start_row(0, 0)
        start_row(1, 1)

        def group(g, acc_s):
            mf = mf_v[pl.ds(g * L, L)]
            for i in range(L):
                b = i % 2
                wait_buf(b)
                rowacc = reduce_row(b)
                start_row(g * L + i + 2, b)
                acc_s = acc_s + rowacc * _bcast_lane(mf, i)
            return acc_s

        acc_s = lax.fori_loop(0, NCH, group, jnp.zeros((L,), jnp.float32))

        cp_t.wait()
        cp_0.wait()

        acc = acc_s * jnp.float32(-_EPS)
        for c in range(NCH):
            t = t_v[pl.ds(c * L, L)]
            g = gt_v[pl.ds(c * L, L)]
            x0 = g0_v[pl.ds(c * L, L)]
            contrib = (jnp.float32(_C_ROW)
                       + jnp.float32(_EPS) * x0
                       + jnp.float32(_EPS - _CONF) * g)
            acc = acc + jnp.where(t != PAD_IDX, contrib, jnp.float32(0.0))

        acc_v[...] = acc
        pltpu.sync_copy(acc_v, out_hbm.at[wid])

    return sc_loss


def _final_body(p_ref, o_ref):
    o_ref[0, 0] = jnp.sum(p_ref[...])


def _final_sum(partials):
    return pl.pallas_call(
        _final_body,
        out_specs=pl.BlockSpec(memory_space=pltpu.SMEM),
        out_shape=jax.ShapeDtypeStruct((1, 1), jnp.float32),
    )(partials)


@jax.jit
def kernel(x, target):
    xf = x.reshape(N_TOKENS * SIZE)
    partials = _build_sc_loss()(xf, target.astype(jnp.int32))
    return _final_sum(partials)[0, 0]


# speedup vs baseline: 1.0833x; 1.0833x over previous
import functools
import math

import jax
import jax.numpy as jnp
from jax import lax
from jax.experimental import pallas as pl
from jax.experimental.pallas import tpu as pltpu
from jax.experimental.pallas import tpu_sc as plsc

SIZE = 32000
PAD_IDX = 0
N_TOKENS = 4096

_SMOOTH = 0.1
_CONF = 1.0 - _SMOOTH
_EPS = _SMOOTH / (SIZE - 2)
_C_ROW = (SIZE - 2) * _EPS * math.log(_EPS) + _CONF * math.log(_CONF)

L = 16
NC = 2
NS = 16
NW = NC * NS
RPW = N_TOKENS // NW
NCH = RPW // L

R_TC = 2048
SPT = (N_TOKENS - R_TC) // NW
NCH_S = SPT // L

_U = 16
_KITERS = SIZE // (L * _U)

_GDN = lax.GatherDimensionNumbers(
    offset_dims=(), collapsed_slice_dims=(0,), start_index_map=(0,))


def _bcast_lane(vec, lane):
    idx = jnp.full((L, 1), lane, jnp.int32)
    return lax.gather(vec, idx, _GDN, slice_sizes=(1,),
                      mode=lax.GatherScatterMode.PROMISE_IN_BOUNDS)


_BR = 128


def _rowsum_body(x_ref, o_ref):
    o_ref[...] = jnp.sum(x_ref[...], axis=1, keepdims=True)


def _rowsums(x):
    return pl.pallas_call(
        _rowsum_body,
        grid=(R_TC // _BR,),
        in_specs=[pl.BlockSpec((_BR, SIZE), lambda r: (r, 0))],
        out_specs=pl.BlockSpec((_BR, 1), lambda r: (r, 0)),
        out_shape=jax.ShapeDtypeStruct((R_TC, 1), jnp.float32),
    )(x)


@functools.lru_cache(maxsize=1)
def _build_sc_loss():
    mesh = plsc.VectorSubcoreMesh(
        core_axis_name="c", subcore_axis_name="s",
        num_cores=NC, num_subcores=NS,
    )

    @functools.partial(
        pl.kernel,
        out_type=jax.ShapeDtypeStruct((NW, L), jnp.float32),
        mesh=mesh,
        scratch_types=[
            pltpu.VMEM((RPW,), jnp.int32),
            pltpu.VMEM((RPW,), jnp.int32),
            pltpu.VMEM((RPW,), jnp.int32),
            pltpu.VMEM((RPW,), jnp.float32),
            pltpu.VMEM((RPW,), jnp.float32),
            pltpu.VMEM((SPT,), jnp.int32),
            pltpu.VMEM((SPT,), jnp.float32),
            pltpu.VMEM((SIZE,), jnp.float32),
            pltpu.VMEM((SIZE,), jnp.float32),
            pltpu.VMEM((L,), jnp.float32),
            pltpu.SemaphoreType.DMA,
            pltpu.SemaphoreType.DMA,
            pltpu.SemaphoreType.DMA,
            pltpu.SemaphoreType.DMA,
        ],
    )
    def sc_loss(xf_hbm, t_hbm, out_hbm,
                t_v, it_v, i0_v, gt_v, g0_v, ts_v, mfs_v, buf0, buf1, acc_v,
                sem_t, sem_0, sem_b0, sem_b1):
        cid = lax.axis_index("c")
        sid = lax.axis_index("s")
        wid = cid * NS + sid
        gbase = wid * RPW
        sbase = R_TC + wid * SPT

        pltpu.sync_copy(t_hbm.at[pl.ds(gbase, RPW)], t_v)
        pltpu.sync_copy(t_hbm.at[pl.ds(sbase, SPT)], ts_v)

        for c in range(NCH):
            t = t_v[pl.ds(c * L, L)]
            rows = (gbase + c * L) + lax.iota(jnp.int32, L)
            flat0 = rows * SIZE
            it_v[pl.ds(c * L, L)] = flat0 + t
            i0_v[pl.ds(c * L, L)] = flat0
        for c in range(NCH_S):
            t = ts_v[pl.ds(c * L, L)]
            mfs_v[pl.ds(c * L, L)] = jnp.where(t != PAD_IDX, jnp.float32(1.0),
                                               jnp.float32(0.0))

        cp_t = pltpu.async_copy(xf_hbm.at[it_v], gt_v, sem_t)
        cp_0 = pltpu.async_copy(xf_hbm.at[i0_v], g0_v, sem_0)

        bufs = (buf0, buf1)
        sems = (sem_b0, sem_b1)

        def start_row(row, b):
            @pl.when(row < SPT)
            def _():
                pltpu.async_copy(
                    xf_hbm.at[pl.ds((sbase + row) * SIZE, SIZE)],
                    bufs[b], sems[b])

        def wait_buf(b):
            pltpu.make_async_copy(xf_hbm.at[pl.ds(0, SIZE)],
                                  bufs[b], sems[b]).wait()

        def reduce_row(b):
            def step(k, accs):
                new = list(accs)
                for u in range(_U):
                    v = bufs[b][pl.ds(k * (L * _U) + u * L, L)]
                    new[u % 4] = new[u % 4] + v
                return tuple(new)

            z = jnp.zeros((L,), jnp.float32)
            a0, a1, a2, a3 = lax.fori_loop(0, _KITERS, step, (z, z, z, z))
            return (a0 + a1) + (a2 + a3)

        start_row(0, 0)
        start_row(1, 1)

        def group(g, acc_s):
            mf = mfs_v[pl.ds(g * L, L)]
            for i in range(L):
                b = i % 2
                wait_buf(b)
                rowacc = reduce_row(b)
                start_row(g * L + i + 2, b)
                acc_s = acc_s + rowacc * _bcast_lane(mf, i)
            return acc_s

        acc_s = lax.fori_loop(0, NCH_S, group, jnp.zeros((L,), jnp.float32))

        cp_t.wait()
        cp_0.wait()

        acc = acc_s * jnp.float32(-_EPS)
        for c in range(NCH):
            t = t_v[pl.ds(c * L, L)]
            g = gt_v[pl.ds(c * L, L)]
            x0 = g0_v[pl.ds(c * L, L)]
            contrib = (jnp.float32(_C_ROW)
                       + jnp.float32(_EPS) * x0
                       + jnp.float32(_EPS - _CONF) * g)
            acc = acc + jnp.where(t != PAD_IDX, contrib, jnp.float32(0.0))

        acc_v[...] = acc
        pltpu.sync_copy(acc_v, out_hbm.at[wid])

    return sc_loss


def _final_body(p_ref, s_ref, t_ref, o_ref):
    masked = jnp.where(t_ref[...] != PAD_IDX, s_ref[...], 0.0)
    o_ref[0, 0] = jnp.sum(p_ref[...]) - jnp.float32(_EPS) * jnp.sum(masked)


def _final_sum(partials, s_tc, t_tc):
    return pl.pallas_call(
        _final_body,
        out_specs=pl.BlockSpec(memory_space=pltpu.SMEM),
        out_shape=jax.ShapeDtypeStruct((1, 1), jnp.float32),
    )(partials, s_tc, t_tc)


@jax.jit
def kernel(x, target):
    tgt = target.astype(jnp.int32)
    xf = x.reshape(N_TOKENS * SIZE)
    partials = _build_sc_loss()(xf, tgt)
    s_tc = _rowsums(x)
    loss = _final_sum(partials, s_tc.reshape(1, R_TC), tgt[:R_TC].reshape(1, R_TC))
    return loss[0, 0]


# speedup vs baseline: 3.1669x; 2.9233x over previous
import functools
import math

import jax
import jax.numpy as jnp
from jax import lax
from jax.experimental import pallas as pl
from jax.experimental.pallas import tpu as pltpu
from jax.experimental.pallas import tpu_sc as plsc

SIZE = 32000
PAD_IDX = 0
N_TOKENS = 4096

_SMOOTH = 0.1
_CONF = 1.0 - _SMOOTH
_EPS = _SMOOTH / (SIZE - 2)
_C_ROW = (SIZE - 2) * _EPS * math.log(_EPS) + _CONF * math.log(_CONF)

L = 16
NC = 2
NS = 16
NW = NC * NS

R_TC = 2560
SPT = (N_TOKENS - R_TC) // NW
NCH_S = SPT // L

_U = 16
_KITERS = SIZE // (L * _U)

_GDN = lax.GatherDimensionNumbers(
    offset_dims=(), collapsed_slice_dims=(0,), start_index_map=(0,))


def _bcast_lane(vec, lane):
    idx = jnp.full((L, 1), lane, jnp.int32)
    return lax.gather(vec, idx, _GDN, slice_sizes=(1,),
                      mode=lax.GatherScatterMode.PROMISE_IN_BOUNDS)


_BR = 128


def _tc_body(x_ref, t_ref, s_ref, g_ref, z_ref):
    xb = x_ref[...]
    tb = t_ref[...]
    colid = lax.broadcasted_iota(jnp.int32, (_BR, SIZE), 1)
    s_ref[...] = jnp.sum(xb, axis=1, keepdims=True)
    g_ref[...] = jnp.sum(jnp.where(colid == tb, xb, 0.0), axis=1,
                         keepdims=True)
    z_ref[...] = xb[:, 0:1]


def _tc_part(x, t2d):
    return pl.pallas_call(
        _tc_body,
        grid=(R_TC // _BR,),
        in_specs=[
            pl.BlockSpec((_BR, SIZE), lambda r: (r, 0)),
            pl.BlockSpec((_BR, 1), lambda r: (r, 0)),
        ],
        out_specs=[
            pl.BlockSpec((_BR, 1), lambda r: (r, 0)),
            pl.BlockSpec((_BR, 1), lambda r: (r, 0)),
            pl.BlockSpec((_BR, 1), lambda r: (r, 0)),
        ],
        out_shape=[
            jax.ShapeDtypeStruct((R_TC, 1), jnp.float32),
            jax.ShapeDtypeStruct((R_TC, 1), jnp.float32),
            jax.ShapeDtypeStruct((R_TC, 1), jnp.float32),
        ],
    )(x, t2d)


@functools.lru_cache(maxsize=1)
def _build_sc_loss():
    mesh = plsc.VectorSubcoreMesh(
        core_axis_name="c", subcore_axis_name="s",
        num_cores=NC, num_subcores=NS,
    )

    @functools.partial(
        pl.kernel,
        out_type=jax.ShapeDtypeStruct((NW, L), jnp.float32),
        mesh=mesh,
        scratch_types=[
            pltpu.VMEM((SPT,), jnp.int32),
            pltpu.VMEM((SPT,), jnp.float32),
            pltpu.VMEM((SIZE,), jnp.float32),
            pltpu.VMEM((SIZE,), jnp.float32),
            pltpu.VMEM((L,), jnp.float32),
            pltpu.SemaphoreType.DMA,
            pltpu.SemaphoreType.DMA,
        ],
    )
    def sc_loss(x_hbm, t_hbm, out_hbm,
                ts_v, mfs_v, buf0, buf1, acc_v, sem_b0, sem_b1):
        cid = lax.axis_index("c")
        sid = lax.axis_index("s")
        wid = cid * NS + sid
        sbase = R_TC + wid * SPT

        pltpu.sync_copy(t_hbm.at[pl.ds(sbase, SPT)], ts_v)
        for c in range(NCH_S):
            t = ts_v[pl.ds(c * L, L)]
            mfs_v[pl.ds(c * L, L)] = jnp.where(t != PAD_IDX, jnp.float32(1.0),
                                               jnp.float32(0.0))

        bufs = (buf0, buf1)
        sems = (sem_b0, sem_b1)
        iota = lax.iota(jnp.int32, L)

        def start_row(row, b):
            @pl.when(row < SPT)
            def _():
                pltpu.async_copy(x_hbm.at[sbase + row], bufs[b], sems[b])

        def wait_buf(b):
            pltpu.make_async_copy(x_hbm.at[0], bufs[b], sems[b]).wait()

        def reduce_row(b):
            def step(k, accs):
                new = list(accs)
                for u in range(_U):
                    v = bufs[b][pl.ds(k * (L * _U) + u * L, L)]
                    new[u % 4] = new[u % 4] + v
                return tuple(new)

            z = jnp.zeros((L,), jnp.float32)
            a0, a1, a2, a3 = lax.fori_loop(0, _KITERS, step, (z, z, z, z))
            return (a0 + a1) + (a2 + a3)

        start_row(0, 0)
        start_row(1, 1)

        def group(g, acc_s):
            mf = mfs_v[pl.ds(g * L, L)]
            tch = ts_v[pl.ds(g * L, L)]
            for i in range(L):
                b = i % 2
                wait_buf(b)
                rowacc = reduce_row(b)
                tj = tch[i]
                vbase = (tj >> 4) << 4
                lane = tj & (L - 1)
                vt = bufs[b][pl.ds(vbase, L)]
                v0 = bufs[b][pl.ds(0, L)]
                extra = (jnp.where(iota == lane,
                                   jnp.float32(_EPS - _CONF) * vt,
                                   jnp.float32(0.0))
                         + jnp.where(iota == 0,
                                     jnp.float32(_C_ROW)
                                     + jnp.float32(_EPS) * v0,
                                     jnp.float32(0.0)))
                start_row(g * L + i + 2, b)
                acc_s = acc_s + _bcast_lane(mf, i) * (
                    extra - jnp.float32(_EPS) * rowacc)
            return acc_s

        acc_s = lax.fori_loop(0, NCH_S, group, jnp.zeros((L,), jnp.float32))

        acc_v[...] = acc_s
        pltpu.sync_copy(acc_v, out_hbm.at[wid])

    return sc_loss


def _final_body(p_ref, s_ref, g_ref, z_ref, t_ref, o_ref):
    m = t_ref[...] != PAD_IDX
    contrib = (jnp.float32(_C_ROW)
               - jnp.float32(_EPS) * s_ref[...]
               + jnp.float32(_EPS) * z_ref[...]
               + jnp.float32(_EPS - _CONF) * g_ref[...])
    o_ref[0, 0] = (jnp.sum(p_ref[...])
                   + jnp.sum(jnp.where(m, contrib, 0.0)))


def _final_sum(partials, s, g, z, t2d):
    return pl.pallas_call(
        _final_body,
        out_specs=pl.BlockSpec(memory_space=pltpu.SMEM),
        out_shape=jax.ShapeDtypeStruct((1, 1), jnp.float32),
    )(partials, s, g, z, t2d)


@jax.jit
def kernel(x, target):
    tgt = target.astype(jnp.int32)
    t2d = tgt[:R_TC].reshape(R_TC, 1)
    partials = _build_sc_loss()(x, tgt)
    s, g, z = _tc_part(x, t2d)
    loss = _final_sum(partials, s, g, z, t2d)
    return loss[0, 0]
